# baseline (speedup 1.0000x reference)
import functools

import jax
import jax.numpy as jnp
from jax.experimental import pallas as pl
from jax.experimental.pallas import tpu as pltpu


def _fill_buf(xb, buf, *, Cin, L):
    buf[0:Cin, 0:1] = jnp.zeros((Cin, 1), buf.dtype)
    buf[0:Cin, 1:L + 1] = xb
    buf[Cin:2 * Cin, 0:L] = xb
    buf[Cin:2 * Cin, L:L + 1] = jnp.zeros((Cin, 1), buf.dtype)


_DG_T = (((0,), (0,)), ((), ()))


def _conv_t(buf, wa_ref, w2_ref, *, Cin, L):
    return (jax.lax.dot_general(buf[:, 0:L], wa_ref[...], _DG_T,
                                preferred_element_type=jnp.float32)
            + jax.lax.dot_general(buf[Cin:2 * Cin, 1:L + 1], w2_ref[...],
                                  _DG_T, preferred_element_type=jnp.float32))


def _stats_kernel(x_ref, wa_ref, w2_ref, s_ref, q_ref, buf, *, Cin, L, nb):
    @pl.when(pl.program_id(1) == 0)
    def _init():
        s_ref[...] = jnp.zeros_like(s_ref)
        q_ref[...] = jnp.zeros_like(q_ref)

    s = jnp.zeros((1, s_ref.shape[2]), jnp.float32)
    q = jnp.zeros((1, q_ref.shape[2]), jnp.float32)
    for b in range(nb):
        _fill_buf(x_ref[b].astype(jnp.bfloat16), buf, Cin=Cin, L=L)
        zt = _conv_t(buf, wa_ref, w2_ref, Cin=Cin, L=L)
        s = s + jnp.sum(zt, axis=0, keepdims=True)
        q = q + jnp.sum(zt * zt, axis=0, keepdims=True)
    s_ref[0] += s
    q_ref[0] += q


def _out_kernel(x_ref, wa_ref, w2_ref, s_ref, q_ref, g_ref, b_ref, o_ref,
                buf, *ys, Cin, L, nb, M, eps, chunk):
    mean = jnp.sum(s_ref[...], axis=0) / M
    var = jnp.maximum(jnp.sum(q_ref[...], axis=0) / M - mean * mean, 0.0)
    invstd = jax.lax.rsqrt(var + eps)
    scale = g_ref[...] * invstd
    shift = b_ref[...] - scale * mean
    for b in range(nb):
        _fill_buf(x_ref[b].astype(jnp.bfloat16), buf, Cin=Cin, L=L)
        zt = _conv_t(buf, wa_ref, w2_ref, Cin=Cin, L=L)
        yt = jnp.maximum(zt * scale + shift, 0.0)
        for c, ysc in enumerate(ys):
            ysc[...] = yt[:, c * chunk:(c + 1) * chunk]
            pc = jnp.maximum(ysc[0:L:2, :], ysc[1:L:2, :])
            o_ref[b, c * chunk:(c + 1) * chunk, :] = pc.T.astype(o_ref.dtype)


def kernel(x_ncl, weight, bias, gamma, beta):
    del bias
    N, Cin, L = x_ncl.shape
    Cout, _, K = weight.shape
    Lh = L // 2
    M = N * L
    eps = 1e-5

    w0 = weight[:, :, 0].astype(jnp.bfloat16)
    w1 = weight[:, :, 1].astype(jnp.bfloat16)
    w2t = weight[:, :, 2].T.astype(jnp.bfloat16)
    wat = jnp.concatenate([w0, w1], axis=1).T
    g2 = gamma.astype(jnp.float32).reshape(1, Cout)
    b2 = beta.astype(jnp.float32).reshape(1, Cout)

    vmem_limit = 60000 * 1024
    NC = 2
    NB1 = 4
    NB = 4
    npc = N // (NC * NB1)
    x_spec1 = pl.BlockSpec((NB1, Cin, L), lambda i, j: (i * npc + j, 0, 0))
    wa_spec1 = pl.BlockSpec((2 * Cin, Cout), lambda i, j: (0, 0))
    w2_spec1 = pl.BlockSpec((Cin, Cout), lambda i, j: (0, 0))
    part_spec = pl.BlockSpec((1, 1, Cout), lambda i, j: (i, 0, 0))

    part_s, part_q = pl.pallas_call(
        functools.partial(_stats_kernel, Cin=Cin, L=L, nb=NB1),
        out_shape=(jax.ShapeDtypeStruct((NC, 1, Cout), jnp.float32),
                   jax.ShapeDtypeStruct((NC, 1, Cout), jnp.float32)),
        grid=(NC, npc),
        in_specs=[x_spec1, wa_spec1, w2_spec1],
        out_specs=(part_spec, part_spec),
        scratch_shapes=[pltpu.VMEM((2 * Cin, L + 1), jnp.bfloat16)],
        compiler_params=pltpu.CompilerParams(
            dimension_semantics=("parallel", "arbitrary"),
            vmem_limit_bytes=vmem_limit),
    )(x_ncl, wat, w2t)

    x_spec2 = pl.BlockSpec((NB, Cin, L), lambda i: (i, 0, 0))
    wa_spec2 = pl.BlockSpec((2 * Cin, Cout), lambda i: (0, 0))
    w2_spec2 = pl.BlockSpec((Cin, Cout), lambda i: (0, 0))
    part_spec2 = pl.BlockSpec((NC, 1, Cout), lambda i: (0, 0, 0))
    ch_spec2 = pl.BlockSpec((1, Cout), lambda i: (0, 0))
    chunk = min(Cout, 128)
    out = pl.pallas_call(
        functools.partial(_out_kernel, Cin=Cin, L=L, nb=NB, M=M, eps=eps,
                          chunk=chunk),
        out_shape=jax.ShapeDtypeStruct((N, Cout, Lh), x_ncl.dtype),
        grid=(N // NB,),
        in_specs=[x_spec2, wa_spec2, w2_spec2, part_spec2, part_spec2,
                  ch_spec2, ch_spec2],
        out_specs=pl.BlockSpec((NB, Cout, Lh), lambda i: (i, 0, 0)),
        scratch_shapes=[pltpu.VMEM((2 * Cin, L + 1), jnp.bfloat16)]
                       + [pltpu.VMEM((L, chunk), jnp.float32)
                          for _ in range(Cout // chunk)],
        compiler_params=pltpu.CompilerParams(
            dimension_semantics=("parallel",),
            vmem_limit_bytes=vmem_limit),
    )(x_ncl, wat, w2t, part_s, part_q, g2, b2)

    return out


# speedup vs baseline: 1.4966x; 1.4966x over previous
import functools

import jax
import jax.numpy as jnp
from jax.experimental import pallas as pl
from jax.experimental.pallas import tpu as pltpu


def _phase_convs(xet, xot, wat_ref, wbt_ref, w0t_ref, w2t_ref, bufe, bufo,
                 *, Cin, Lh):
    bufe[0:1, 0:Cin] = jnp.zeros((1, Cin), bufe.dtype)
    bufe[1:Lh, 0:Cin] = xot[0:Lh - 1]
    bufe[:, Cin:2 * Cin] = xet
    bufo[:, 0:Cin] = xot
    bufo[0:Lh - 1, Cin:2 * Cin] = xet[1:Lh]
    bufo[Lh - 1:Lh, Cin:2 * Cin] = jnp.zeros((1, Cin), bufo.dtype)
    zte = (jnp.dot(bufe[...], wat_ref[...],
                   preferred_element_type=jnp.float32)
           + jnp.dot(xot, w2t_ref[...], preferred_element_type=jnp.float32))
    zto = (jnp.dot(bufo[...], wbt_ref[...],
                   preferred_element_type=jnp.float32)
           + jnp.dot(xet, w0t_ref[...], preferred_element_type=jnp.float32))
    return zte, zto


def _stats_kernel(x_ref, wat_ref, wbt_ref, w0t_ref, w2t_ref,
                  x2t_ref, s_ref, q_ref, xts, bufe, bufo,
                  *, Cin, L, nb):
    Lh = L // 2

    @pl.when(pl.program_id(1) == 0)
    def _init():
        s_ref[...] = jnp.zeros_like(s_ref)
        q_ref[...] = jnp.zeros_like(q_ref)

    s = jnp.zeros((1, s_ref.shape[2]), jnp.float32)
    q = jnp.zeros((1, q_ref.shape[2]), jnp.float32)
    for b in range(nb):
        xts[...] = x_ref[b].T
        xet = xts[0:L:2, :].astype(jnp.bfloat16)
        xot = xts[1:L:2, :].astype(jnp.bfloat16)
        x2t_ref[b, 0] = xet
        x2t_ref[b, 1] = xot
        zte, zto = _phase_convs(xet, xot, wat_ref, wbt_ref, w0t_ref, w2t_ref,
                                bufe, bufo, Cin=Cin, Lh=Lh)
        s = s + (jnp.sum(zte, axis=0, keepdims=True)
                 + jnp.sum(zto, axis=0, keepdims=True))
        q = q + (jnp.sum(zte * zte, axis=0, keepdims=True)
                 + jnp.sum(zto * zto, axis=0, keepdims=True))
    s_ref[0] += s
    q_ref[0] += q


def _out_kernel(x2t_ref, wat_ref, wbt_ref, w0t_ref, w2t_ref,
                s_ref, q_ref, g_ref, b_ref, o_ref, bufe, bufo,
                *, Cin, Lh, nb, M, eps):
    mean = jnp.sum(s_ref[...], axis=0) / M
    var = jnp.maximum(jnp.sum(q_ref[...], axis=0) / M - mean * mean, 0.0)
    invstd = jax.lax.rsqrt(var + eps)
    scale = g_ref[...] * invstd
    shift = b_ref[...] - scale * mean
    for b in range(nb):
        zte, zto = _phase_convs(x2t_ref[b, 0], x2t_ref[b, 1],
                                wat_ref, wbt_ref, w0t_ref, w2t_ref,
                                bufe, bufo, Cin=Cin, Lh=Lh)
        yte = jnp.maximum(zte * scale + shift, 0.0)
        yto = jnp.maximum(zto * scale + shift, 0.0)
        o_ref[b] = jnp.maximum(yte, yto).T.astype(o_ref.dtype)


def kernel(x_ncl, weight, bias, gamma, beta):
    del bias
    N, Cin, L = x_ncl.shape
    Cout, _, K = weight.shape
    Lh = L // 2
    M = N * L
    eps = 1e-5

    w0 = weight[:, :, 0].astype(jnp.bfloat16)
    w1 = weight[:, :, 1].astype(jnp.bfloat16)
    w2 = weight[:, :, 2].astype(jnp.bfloat16)
    wat = jnp.concatenate([w0, w1], axis=1).T
    wbt = jnp.concatenate([w1, w2], axis=1).T
    w0t = w0.T
    w2t = w2.T
    g2 = gamma.astype(jnp.float32).reshape(1, Cout)
    b2 = beta.astype(jnp.float32).reshape(1, Cout)

    vmem_limit = 60000 * 1024
    NC = 2
    NB1 = 4
    NB = 4
    npc = N // (NC * NB1)
    x_spec1 = pl.BlockSpec((NB1, Cin, L), lambda i, j: (i * npc + j, 0, 0))
    wab_spec1 = pl.BlockSpec((2 * Cin, Cout), lambda i, j: (0, 0))
    w_spec1 = pl.BlockSpec((Cin, Cout), lambda i, j: (0, 0))
    x2t_spec1 = pl.BlockSpec((NB1, 2, Lh, Cin),
                             lambda i, j: (i * npc + j, 0, 0, 0))
    part_spec = pl.BlockSpec((1, 1, Cout), lambda i, j: (i, 0, 0))

    x2t, part_s, part_q = pl.pallas_call(
        functools.partial(_stats_kernel, Cin=Cin, L=L, nb=NB1),
        out_shape=(jax.ShapeDtypeStruct((N, 2, Lh, Cin), jnp.bfloat16),
                   jax.ShapeDtypeStruct((NC, 1, Cout), jnp.float32),
                   jax.ShapeDtypeStruct((NC, 1, Cout), jnp.float32)),
        grid=(NC, npc),
        in_specs=[x_spec1, wab_spec1, wab_spec1, w_spec1, w_spec1],
        out_specs=(x2t_spec1, part_spec, part_spec),
        scratch_shapes=[pltpu.VMEM((L, Cin), jnp.float32),
                        pltpu.VMEM((Lh, 2 * Cin), jnp.bfloat16),
                        pltpu.VMEM((Lh, 2 * Cin), jnp.bfloat16)],
        compiler_params=pltpu.CompilerParams(
            dimension_semantics=("parallel", "arbitrary"),
            vmem_limit_bytes=vmem_limit),
    )(x_ncl, wat, wbt, w0t, w2t)

    x2t_spec2 = pl.BlockSpec((NB, 2, Lh, Cin), lambda i: (i, 0, 0, 0))
    wab_spec2 = pl.BlockSpec((2 * Cin, Cout), lambda i: (0, 0))
    w_spec2 = pl.BlockSpec((Cin, Cout), lambda i: (0, 0))
    part_spec2 = pl.BlockSpec((NC, 1, Cout), lambda i: (0, 0, 0))
    ch_spec2 = pl.BlockSpec((1, Cout), lambda i: (0, 0))
    out = pl.pallas_call(
        functools.partial(_out_kernel, Cin=Cin, Lh=Lh, nb=NB, M=M, eps=eps),
        out_shape=jax.ShapeDtypeStruct((N, Cout, Lh), x_ncl.dtype),
        grid=(N // NB,),
        in_specs=[x2t_spec2, wab_spec2, wab_spec2, w_spec2, w_spec2,
                  part_spec2, part_spec2, ch_spec2, ch_spec2],
        out_specs=pl.BlockSpec((NB, Cout, Lh), lambda i: (i, 0, 0)),
        scratch_shapes=[pltpu.VMEM((Lh, 2 * Cin), jnp.bfloat16),
                        pltpu.VMEM((Lh, 2 * Cin), jnp.bfloat16)],
        compiler_params=pltpu.CompilerParams(
            dimension_semantics=("parallel",),
            vmem_limit_bytes=vmem_limit),
    )(x2t, wat, wbt, w0t, w2t, part_s, part_q, g2, b2)

    return out
